# baseline (speedup 1.0000x reference)
import functools

import jax
import jax.numpy as jnp
from jax import lax
from jax.experimental import pallas as pl
from jax.experimental.pallas import tpu as pltpu
from jax.experimental.pallas import tpu_sc as plsc

N = 100000
D = 128
S = 512
NC = 2
NS = 16
NW = NC * NS
BLK = 128
NB = N // BLK
TAIL = N - NB * BLK
SEG_PER_TILE = S // NS
BASE_BLOCKS = NB // NW
EXTRA_WORKERS = NB - BASE_BLOCKS * NW
MAXB = BASE_BLOCKS + 1
MCH = 32


def _sc_partials(x, batch):
    mesh = plsc.VectorSubcoreMesh(core_axis_name="c", subcore_axis_name="s")

    @functools.partial(
        pl.kernel,
        out_type=[
            jax.ShapeDtypeStruct((NC, S, D), jnp.float32),
            jax.ShapeDtypeStruct((NW, S), jnp.float32),
        ],
        mesh=mesh,
        compiler_params=pltpu.CompilerParams(needs_layout_passes=False,
                                             use_tc_tiling_on_sc=False),
        scratch_types=[
            pltpu.VMEM((2, BLK, D), jnp.float32),
            pltpu.VMEM((MAXB, BLK), jnp.int32),
            pltpu.VMEM((S, D), jnp.float32),
            pltpu.VMEM((TAIL, D), jnp.float32),
            pltpu.VMEM((TAIL,), jnp.int32),
            pltpu.VMEM((MCH,), jnp.int32),
            pltpu.VMEM((S,), jnp.float32),
            pltpu.VMEM((MCH, D), jnp.float32),
            pltpu.VMEM_SHARED((S, D), jnp.float32),
            pltpu.SemaphoreType.DMA((2,)),
            pltpu.SemaphoreType.DMA,
            pltpu.SemaphoreType.DMA,
        ],
    )
    def sc_kernel(x_hbm, b_hbm, sum_out, cnt_out,
                  xbufs, ids_all, acc_loc, xt, ids_t, idxbuf, hist, zbuf, acc,
                  ld_sems, sc_sem, id_sem):
        c = lax.axis_index("c")
        s = lax.axis_index("s")
        wid = s * NC + c

        sb = BASE_BLOCKS * wid + jnp.minimum(wid, EXTRA_WORKERS)
        nblk = BASE_BLOCKS + jnp.where(wid < EXTRA_WORKERS, 1, 0)

        for k in range(MAXB):
            @pl.when(k < nblk)
            def _stage_ids():
                pltpu.async_copy(b_hbm.at[pl.ds((sb + k) * BLK, BLK)],
                                 ids_all.at[k], id_sem)

        for p in range(2):
            pltpu.async_copy(x_hbm.at[pl.ds((sb + p) * BLK, BLK)],
                             xbufs.at[p], ld_sems.at[p])

        zeros16 = jnp.zeros((16,), jnp.float32)

        def zrow(i, carry):
            def zcol(j, carry2):
                zbuf[i, pl.ds(j * 16, 16)] = zeros16
                return carry2
            return lax.fori_loop(0, D // 16, zcol, carry)
        lax.fori_loop(0, MCH, zrow, 0)

        def zh(i, carry):
            hist[pl.ds(i * 16, 16)] = zeros16
            return carry
        lax.fori_loop(0, S // 16, zh, 0)

        pltpu.sync_copy(zbuf, acc.at[pl.ds(s * SEG_PER_TILE, SEG_PER_TILE)])
        plsc.subcore_barrier()

        for k in range(MAXB):
            @pl.when(k < nblk)
            def _drain_ids():
                pltpu.make_async_copy(b_hbm.at[pl.ds((sb + k) * BLK, BLK)],
                                      ids_all.at[k], id_sem).wait()

        fs = jnp.min(ids_all[0, pl.ds(0, 16)])
        ls = jnp.max(ids_all[nblk - 1, pl.ds(BLK - 16, 16)])

        def zpriv(r, carry):
            for j in range(D // 16):
                acc_loc[r, pl.ds(j * 16, 16)] = zeros16
            return carry
        lax.fori_loop(fs, ls + 1, zpriv, 0)

        lane = lax.iota(jnp.int32, 16)
        ones = jnp.full((16,), 1.0, jnp.float32)

        def hist_block(id_row_ref, k, nvec):
            def grp(g, carry2):
                idv = id_row_ref[k, pl.ds(g * 16, 16)]
                plsc.addupdate_scatter(hist, [idv], ones)
                return carry2
            lax.fori_loop(0, nvec, grp, 0)

        def pair(i, carry):
            k0 = 2 * i
            k1 = 2 * i + 1

            @pl.when(k0 < nblk)
            def _stream_block():
                pltpu.make_async_copy(
                    x_hbm.at[pl.ds((sb + k0) * BLK, BLK)],
                    xbufs.at[0], ld_sems.at[0]).wait()
                pltpu.async_copy(xbufs.at[0], acc.at[ids_all.at[k0]],
                                 sc_sem, add=True)
                hist_block(ids_all, k0, BLK // 16)

            @pl.when(k1 < nblk)
            def _tec_block():
                pltpu.make_async_copy(
                    x_hbm.at[pl.ds((sb + k1) * BLK, BLK)],
                    xbufs.at[1], ld_sems.at[1]).wait()
                hist_block(ids_all, k1, BLK // 16)

                def grp(g, carry2):
                    idv = ids_all[k1, pl.ds(g * 16, 16)]
                    for m in range(16):
                        bid = idv[m]
                        r = g * 16 + m
                        for j in range(D // 16):
                            plsc.addupdate(
                                acc_loc.at[bid, pl.ds(j * 16, 16)],
                                xbufs[1, r, pl.ds(j * 16, 16)])
                    return carry2
                lax.fori_loop(0, BLK // 16, grp, 0)

                @pl.when(k1 + 2 < nblk)
                def _next_load1():
                    pltpu.async_copy(
                        x_hbm.at[pl.ds((sb + k1 + 2) * BLK, BLK)],
                        xbufs.at[1], ld_sems.at[1])

            @pl.when(k0 < nblk)
            def _drain_stream():
                pltpu.make_async_copy(xbufs.at[0], acc.at[ids_all.at[k0]],
                                      sc_sem).wait()

                @pl.when(k0 + 2 < nblk)
                def _next_load0():
                    pltpu.async_copy(
                        x_hbm.at[pl.ds((sb + k0 + 2) * BLK, BLK)],
                        xbufs.at[0], ld_sems.at[0])
            return carry
        lax.fori_loop(0, (MAXB + 1) // 2, pair, 0)

        @pl.when(wid == NW - 1)
        def _tail():
            base = NB * BLK
            pltpu.sync_copy(b_hbm.at[pl.ds(base, TAIL)], ids_t)
            pltpu.sync_copy(x_hbm.at[pl.ds(base, TAIL)], xt)
            pltpu.sync_copy(xt, acc.at[ids_t], add=True)

            def grp(g, carry):
                idv = ids_t[pl.ds(g * 16, 16)]
                plsc.addupdate_scatter(hist, [idv], ones)
                return carry
            lax.fori_loop(0, TAIL // 16, grp, 0)

        c0 = fs // MCH
        c1 = ls // MCH
        for ci in range(S // MCH):
            @pl.when((ci >= c0) & (ci <= c1))
            def _merge_chunk():
                idxbuf[pl.ds(0, 16)] = ci * MCH + lane
                idxbuf[pl.ds(16, 16)] = ci * MCH + 16 + lane
                pltpu.sync_copy(acc_loc.at[pl.ds(ci * MCH, MCH)],
                                acc.at[idxbuf], add=True)

        pltpu.sync_copy(hist, cnt_out.at[wid])
        plsc.subcore_barrier()
        pltpu.sync_copy(acc.at[pl.ds(s * SEG_PER_TILE, SEG_PER_TILE)],
                        sum_out.at[c, pl.ds(s * SEG_PER_TILE, SEG_PER_TILE)])

    return sc_kernel(x, batch)


def _combine(partial_sums, partial_counts):
    def body(sp_ref, cn_ref, o_ref):
        total = sp_ref[0] + sp_ref[1]
        cnt = jnp.maximum(jnp.sum(cn_ref[...], axis=0), 1.0)
        o_ref[...] = total / cnt[:, None]

    return pl.pallas_call(
        body,
        out_shape=jax.ShapeDtypeStruct((S, D), jnp.float32),
    )(partial_sums, partial_counts)


def kernel(x, batch):
    batch = batch.astype(jnp.int32)
    partial_sums, partial_counts = _sc_partials(x, batch)
    return _combine(partial_sums, partial_counts)


# speedup vs baseline: 1.6433x; 1.6433x over previous
import functools

import jax
import jax.numpy as jnp
from jax import lax
from jax.experimental import pallas as pl
from jax.experimental.pallas import tpu as pltpu
from jax.experimental.pallas import tpu_sc as plsc

N = 100000
D = 128
S = 512
NC = 2
NS = 16
NW = NC * NS
BLK = 128
NB = N // BLK
TAIL = N - NB * BLK
SEG_PER_TILE = S // NS
BASE_BLOCKS = NB // NW
EXTRA_WORKERS = NB - BASE_BLOCKS * NW
MAXB = BASE_BLOCKS + 1
MAXP = (MAXB + 1) // 2


def _sc_partials(x, batch):
    mesh = plsc.VectorSubcoreMesh(core_axis_name="c", subcore_axis_name="s")

    @functools.partial(
        pl.kernel,
        out_type=[
            jax.ShapeDtypeStruct((NC, S, D), jnp.float32),
            jax.ShapeDtypeStruct((NW, S), jnp.float32),
        ],
        mesh=mesh,
        compiler_params=pltpu.CompilerParams(needs_layout_passes=False,
                                             use_tc_tiling_on_sc=False),
        scratch_types=[
            pltpu.VMEM((2, 2 * BLK, D), jnp.float32),
            pltpu.VMEM((MAXB, BLK), jnp.int32),
            pltpu.VMEM((TAIL, D), jnp.float32),
            pltpu.VMEM((TAIL,), jnp.int32),
            pltpu.VMEM((S,), jnp.float32),
            pltpu.VMEM((SEG_PER_TILE, D), jnp.float32),
            pltpu.VMEM_SHARED((S, D), jnp.float32),
            pltpu.SemaphoreType.DMA((2,)),
            pltpu.SemaphoreType.DMA((2,)),
            pltpu.SemaphoreType.DMA,
        ],
    )
    def sc_kernel(x_hbm, b_hbm, sum_out, cnt_out,
                  xbufs, ids_all, xt, ids_t, hist, zbuf, acc,
                  ld_sems, sc_sems, id_sem):
        c = lax.axis_index("c")
        s = lax.axis_index("s")
        wid = s * NC + c

        sb = BASE_BLOCKS * wid + jnp.minimum(wid, EXTRA_WORKERS)
        nblk = BASE_BLOCKS + jnp.where(wid < EXTRA_WORKERS, 1, 0)

        for k in range(MAXB):
            @pl.when(k < nblk)
            def _stage_ids():
                pltpu.async_copy(b_hbm.at[pl.ds((sb + k) * BLK, BLK)],
                                 ids_all.at[k], id_sem)

        for b in range(2):
            pltpu.async_copy(x_hbm.at[pl.ds((sb + 2 * b) * BLK, 2 * BLK)],
                             xbufs.at[b], ld_sems.at[b])

        zeros16 = jnp.zeros((16,), jnp.float32)

        def zrow(i, carry):
            def zcol(j, carry2):
                zbuf[i, pl.ds(j * 16, 16)] = zeros16
                return carry2
            return lax.fori_loop(0, D // 16, zcol, carry)
        lax.fori_loop(0, SEG_PER_TILE, zrow, 0)

        def zh(i, carry):
            hist[pl.ds(i * 16, 16)] = zeros16
            return carry
        lax.fori_loop(0, S // 16, zh, 0)

        pltpu.sync_copy(zbuf, acc.at[pl.ds(s * SEG_PER_TILE, SEG_PER_TILE)])
        plsc.subcore_barrier()

        for k in range(MAXB):
            @pl.when(k < nblk)
            def _drain_ids():
                pltpu.make_async_copy(b_hbm.at[pl.ds((sb + k) * BLK, BLK)],
                                      ids_all.at[k], id_sem).wait()

        lane = lax.iota(jnp.int32, 16)
        ones = jnp.full((16,), 1.0, jnp.float32)
        np_ = (nblk + 1) // 2

        def hist_block(k):
            def grp(g, carry2):
                idv = ids_all[k, pl.ds(g * 16, 16)]
                plsc.addupdate_scatter(hist, [idv], ones)
                return carry2
            lax.fori_loop(0, BLK // 16, grp, 0)

        def issue_pair(b, i):
            k0 = 2 * i
            k1 = 2 * i + 1
            pltpu.async_copy(xbufs.at[b, pl.ds(0, BLK)],
                             acc.at[ids_all.at[k0]], sc_sems.at[b], add=True)

            @pl.when(k1 < nblk)
            def _second():
                pltpu.async_copy(xbufs.at[b, pl.ds(BLK, BLK)],
                                 acc.at[ids_all.at[k1]], sc_sems.at[b],
                                 add=True)
            hist_block(k0)

            @pl.when(k1 < nblk)
            def _hist2():
                hist_block(k1)

        def drain_pair(b, i):
            pltpu.make_async_copy(xbufs.at[b, pl.ds(0, BLK)],
                                  acc.at[ids_all.at[2 * i]],
                                  sc_sems.at[b]).wait()

            @pl.when(2 * i + 1 < nblk)
            def _d2():
                pltpu.make_async_copy(xbufs.at[b, pl.ds(BLK, BLK)],
                                      acc.at[ids_all.at[2 * i + 1]],
                                      sc_sems.at[b]).wait()

        def spair(ii, carry):
            for b in range(2):
                i = 2 * ii + b

                @pl.when(i < np_)
                def _pair():
                    pltpu.make_async_copy(
                        x_hbm.at[pl.ds((sb + 2 * i) * BLK, 2 * BLK)],
                        xbufs.at[b], ld_sems.at[b]).wait()
                    issue_pair(b, i)

                    @pl.when(i >= 1)
                    def _prev():
                        drain_pair(1 - b, i - 1)

                        @pl.when(i + 1 < np_)
                        def _next_load():
                            pltpu.async_copy(
                                x_hbm.at[pl.ds((sb + 2 * (i + 1)) * BLK,
                                               2 * BLK)],
                                xbufs.at[1 - b], ld_sems.at[1 - b])
            return carry
        lax.fori_loop(0, (MAXP + 1) // 2, spair, 0)

        for b in range(2):
            @pl.when(((np_ - 1) % 2) == b)
            def _final_drain():
                drain_pair(b, np_ - 1)

        @pl.when(wid == NW - 1)
        def _tail():
            base = NB * BLK
            pltpu.sync_copy(b_hbm.at[pl.ds(base, TAIL)], ids_t)
            pltpu.sync_copy(x_hbm.at[pl.ds(base, TAIL)], xt)
            pltpu.sync_copy(xt, acc.at[ids_t], add=True)

            def grp(g, carry):
                idv = ids_t[pl.ds(g * 16, 16)]
                plsc.addupdate_scatter(hist, [idv], ones)
                return carry
            lax.fori_loop(0, TAIL // 16, grp, 0)

        pltpu.sync_copy(hist, cnt_out.at[wid])
        plsc.subcore_barrier()
        pltpu.sync_copy(acc.at[pl.ds(s * SEG_PER_TILE, SEG_PER_TILE)],
                        sum_out.at[c, pl.ds(s * SEG_PER_TILE, SEG_PER_TILE)])

    return sc_kernel(x, batch)


def _combine(partial_sums, partial_counts):
    def body(sp_ref, cn_ref, o_ref):
        total = sp_ref[0] + sp_ref[1]
        cnt = jnp.maximum(jnp.sum(cn_ref[...], axis=0), 1.0)
        o_ref[...] = total / cnt[:, None]

    return pl.pallas_call(
        body,
        out_shape=jax.ShapeDtypeStruct((S, D), jnp.float32),
    )(partial_sums, partial_counts)


def kernel(x, batch):
    batch = batch.astype(jnp.int32)
    partial_sums, partial_counts = _sc_partials(x, batch)
    return _combine(partial_sums, partial_counts)


# speedup vs baseline: 1.7423x; 1.0603x over previous
import functools

import jax
import jax.numpy as jnp
from jax import lax
from jax.experimental import pallas as pl
from jax.experimental.pallas import tpu as pltpu
from jax.experimental.pallas import tpu_sc as plsc

N = 100000
D = 128
S = 512
NC = 2
NS = 16
NW = NC * NS
BLK = 128
NB = N // BLK
TAIL = N - NB * BLK
SEG_PER_TILE = S // NS
BASE_BLOCKS = NB // NW
EXTRA_WORKERS = NB - BASE_BLOCKS * NW
MAXB = BASE_BLOCKS + 1


def _sc_partials(x, batch):
    mesh = plsc.VectorSubcoreMesh(core_axis_name="c", subcore_axis_name="s")

    @functools.partial(
        pl.kernel,
        out_type=[
            jax.ShapeDtypeStruct((NC * S, D), jnp.float32),
            jax.ShapeDtypeStruct((NW, S), jnp.float32),
        ],
        mesh=mesh,
        compiler_params=pltpu.CompilerParams(needs_layout_passes=False,
                                             use_tc_tiling_on_sc=False),
        scratch_types=[
            pltpu.VMEM((2, BLK, D), jnp.float32),
            pltpu.VMEM((MAXB, BLK), jnp.int32),
            pltpu.VMEM((TAIL, D), jnp.float32),
            pltpu.VMEM((TAIL,), jnp.int32),
            pltpu.VMEM((S,), jnp.float32),
            pltpu.VMEM((SEG_PER_TILE, D), jnp.float32),
            pltpu.VMEM_SHARED((S, D), jnp.float32),
            pltpu.SemaphoreType.DMA((2,)),
            pltpu.SemaphoreType.DMA,
            pltpu.SemaphoreType.DMA,
        ],
    )
    def sc_kernel(x_hbm, b_hbm, sum_out, cnt_out,
                  xbufs, ids_all, xt, ids_t, hist, zbuf, acc,
                  ld_sems, sc_sem, id_sem):
        c = lax.axis_index("c")
        s = lax.axis_index("s")
        wid = s * NC + c

        sb = BASE_BLOCKS * wid + jnp.minimum(wid, EXTRA_WORKERS)
        nblk = BASE_BLOCKS + jnp.where(wid < EXTRA_WORKERS, 1, 0)

        for k in range(MAXB):
            @pl.when(k < nblk)
            def _stage_ids():
                pltpu.async_copy(b_hbm.at[pl.ds((sb + k) * BLK, BLK)],
                                 ids_all.at[k], id_sem)

        for p in range(2):
            pltpu.async_copy(x_hbm.at[pl.ds((sb + p) * BLK, BLK)],
                             xbufs.at[p], ld_sems.at[p])

        zeros16 = jnp.zeros((16,), jnp.float32)

        def zrow(i, carry):
            def zcol(j, carry2):
                zbuf[i, pl.ds(j * 16, 16)] = zeros16
                return carry2
            return lax.fori_loop(0, D // 16, zcol, carry)
        lax.fori_loop(0, SEG_PER_TILE, zrow, 0)

        def zh(i, carry):
            hist[pl.ds(i * 16, 16)] = zeros16
            return carry
        lax.fori_loop(0, S // 16, zh, 0)

        pltpu.sync_copy(zbuf, acc.at[pl.ds(s * SEG_PER_TILE, SEG_PER_TILE)])
        plsc.subcore_barrier()

        for k in range(MAXB):
            @pl.when(k < nblk)
            def _drain_ids():
                pltpu.make_async_copy(b_hbm.at[pl.ds((sb + k) * BLK, BLK)],
                                      ids_all.at[k], id_sem).wait()

        lane = lax.iota(jnp.int32, 16)
        ones = jnp.full((16,), 1.0, jnp.float32)

        def pair(i, carry):
            for p in range(2):
                k = 2 * i + p

                @pl.when(k < nblk)
                def _block():
                    pltpu.make_async_copy(
                        x_hbm.at[pl.ds((sb + k) * BLK, BLK)],
                        xbufs.at[p], ld_sems.at[p]).wait()
                    h = pltpu.async_copy(xbufs.at[p], acc.at[ids_all.at[k]],
                                         sc_sem, add=True)

                    def grp(g, carry2):
                        idv = ids_all[k, pl.ds(g * 16, 16)]
                        plsc.addupdate_scatter(hist, [idv], ones)
                        return carry2
                    lax.fori_loop(0, BLK // 16, grp, 0)
                    h.wait()

                    @pl.when(k + 2 < nblk)
                    def _next_load():
                        pltpu.async_copy(
                            x_hbm.at[pl.ds((sb + k + 2) * BLK, BLK)],
                            xbufs.at[p], ld_sems.at[p])
            return carry
        lax.fori_loop(0, (MAXB + 1) // 2, pair, 0)

        @pl.when(wid == NW - 1)
        def _tail():
            base = NB * BLK
            pltpu.sync_copy(b_hbm.at[pl.ds(base, TAIL)], ids_t)
            pltpu.sync_copy(x_hbm.at[pl.ds(base, TAIL)], xt)
            pltpu.sync_copy(xt, acc.at[ids_t], add=True)

            def grp(g, carry):
                idv = ids_t[pl.ds(g * 16, 16)]
                plsc.addupdate_scatter(hist, [idv], ones)
                return carry
            lax.fori_loop(0, TAIL // 16, grp, 0)

        pltpu.sync_copy(hist, cnt_out.at[wid])
        plsc.subcore_barrier()
        pltpu.sync_copy(acc.at[pl.ds(s * SEG_PER_TILE, SEG_PER_TILE)],
                        sum_out.at[pl.ds(c * S + s * SEG_PER_TILE,
                                         SEG_PER_TILE)])

    return sc_kernel(x, batch)


def _combine(partial_sums, partial_counts):
    def body(sp_ref, cn_ref, o_ref):
        total = sp_ref[pl.ds(0, S)] + sp_ref[pl.ds(S, S)]
        cnt = jnp.maximum(jnp.sum(cn_ref[...], axis=0), 1.0)
        o_ref[...] = total / cnt[:, None]

    return pl.pallas_call(
        body,
        out_shape=jax.ShapeDtypeStruct((S, D), jnp.float32),
    )(partial_sums, partial_counts)


def kernel(x, batch):
    batch = batch.astype(jnp.int32)
    partial_sums, partial_counts = _sc_partials(x, batch)
    return _combine(partial_sums, partial_counts)


# speedup vs baseline: 1.7440x; 1.0009x over previous
import functools

import jax
import jax.numpy as jnp
from jax import lax
from jax.experimental import pallas as pl
from jax.experimental.pallas import tpu as pltpu
from jax.experimental.pallas import tpu_sc as plsc

N = 100000
D = 128
S = 512
NC = 2
NS = 16
NW = NC * NS
BLK = 128
NB = N // BLK
TAIL = N - NB * BLK
SEG_PER_TILE = S // NS
BASE_BLOCKS = NB // NW
EXTRA_WORKERS = NB - BASE_BLOCKS * NW
MAXB = BASE_BLOCKS + 1


def _sc_partials(x, batch):
    mesh = plsc.VectorSubcoreMesh(core_axis_name="c", subcore_axis_name="s")

    @functools.partial(
        pl.kernel,
        out_type=[
            jax.ShapeDtypeStruct((NC * S, D), jnp.float32),
            jax.ShapeDtypeStruct((NW, S), jnp.float32),
        ],
        mesh=mesh,
        compiler_params=pltpu.CompilerParams(needs_layout_passes=False,
                                             use_tc_tiling_on_sc=False),
        scratch_types=[
            pltpu.VMEM((2, BLK, D), jnp.float32),
            pltpu.VMEM((MAXB, BLK), jnp.int32),
            pltpu.VMEM((TAIL, D), jnp.float32),
            pltpu.VMEM((TAIL,), jnp.int32),
            pltpu.VMEM((S,), jnp.float32),
            pltpu.VMEM((SEG_PER_TILE, D), jnp.float32),
            pltpu.VMEM_SHARED((S, D), jnp.float32),
            pltpu.SemaphoreType.DMA((2,)),
            pltpu.SemaphoreType.DMA,
            pltpu.SemaphoreType.DMA,
        ],
    )
    def sc_kernel(x_hbm, b_hbm, sum_out, cnt_out,
                  xbufs, ids_all, xt, ids_t, hist, zbuf, acc,
                  ld_sems, sc_sem, id_sem):
        c = lax.axis_index("c")
        s = lax.axis_index("s")
        wid = s * NC + c

        sb = BASE_BLOCKS * wid + jnp.minimum(wid, EXTRA_WORKERS)
        nblk = BASE_BLOCKS + jnp.where(wid < EXTRA_WORKERS, 1, 0)

        for k in range(MAXB):
            @pl.when(k < nblk)
            def _stage_ids():
                pltpu.async_copy(b_hbm.at[pl.ds((sb + k) * BLK, BLK)],
                                 ids_all.at[k], id_sem)

        for p in range(2):
            pltpu.async_copy(x_hbm.at[pl.ds((sb + p) * BLK, BLK)],
                             xbufs.at[p], ld_sems.at[p])

        zeros16 = jnp.zeros((16,), jnp.float32)

        def zrow(i, carry):
            def zcol(j, carry2):
                zbuf[i, pl.ds(j * 16, 16)] = zeros16
                return carry2
            return lax.fori_loop(0, D // 16, zcol, carry)
        lax.fori_loop(0, SEG_PER_TILE, zrow, 0)

        def zh(i, carry):
            hist[pl.ds(i * 16, 16)] = zeros16
            return carry
        lax.fori_loop(0, S // 16, zh, 0)

        pltpu.sync_copy(zbuf, acc.at[pl.ds(s * SEG_PER_TILE, SEG_PER_TILE)])
        plsc.subcore_barrier()

        for k in range(MAXB):
            @pl.when(k < nblk)
            def _drain_ids():
                pltpu.make_async_copy(b_hbm.at[pl.ds((sb + k) * BLK, BLK)],
                                      ids_all.at[k], id_sem).wait()

        ones = jnp.full((16,), 1.0, jnp.float32)

        def pair(i, carry):
            for p in range(2):
                k = 2 * i + p

                @pl.when(k < nblk)
                def _block():
                    pltpu.make_async_copy(
                        x_hbm.at[pl.ds((sb + k) * BLK, BLK)],
                        xbufs.at[p], ld_sems.at[p]).wait()
                    h = pltpu.async_copy(xbufs.at[p], acc.at[ids_all.at[k]],
                                         sc_sem, add=True)

                    def grp(g, carry2):
                        idv = ids_all[k, pl.ds(g * 16, 16)]
                        plsc.addupdate_scatter(hist, [idv], ones)
                        return carry2
                    lax.fori_loop(0, BLK // 16, grp, 0)
                    h.wait()

                    @pl.when(k + 2 < nblk)
                    def _next_load():
                        pltpu.async_copy(
                            x_hbm.at[pl.ds((sb + k + 2) * BLK, BLK)],
                            xbufs.at[p], ld_sems.at[p])
            return carry
        lax.fori_loop(0, (MAXB + 1) // 2, pair, 0)

        @pl.when(wid == NW - 1)
        def _tail():
            base = NB * BLK
            pltpu.sync_copy(b_hbm.at[pl.ds(base, TAIL)], ids_t)
            pltpu.sync_copy(x_hbm.at[pl.ds(base, TAIL)], xt)
            pltpu.sync_copy(xt, acc.at[ids_t], add=True)

            def grp(g, carry):
                idv = ids_t[pl.ds(g * 16, 16)]
                plsc.addupdate_scatter(hist, [idv], ones)
                return carry
            lax.fori_loop(0, TAIL // 16, grp, 0)

        pltpu.sync_copy(hist, cnt_out.at[wid])
        plsc.subcore_barrier()
        pltpu.sync_copy(acc.at[pl.ds(s * SEG_PER_TILE, SEG_PER_TILE)],
                        sum_out.at[pl.ds(c * S + s * SEG_PER_TILE,
                                         SEG_PER_TILE)])

    return sc_kernel(x, batch)


def _combine(partial_sums, partial_counts):
    def body(sp_ref, cn_ref, o_ref):
        total = sp_ref[pl.ds(0, S)] + sp_ref[pl.ds(S, S)]
        cnt = jnp.maximum(jnp.sum(cn_ref[...], axis=0), 1.0)
        o_ref[...] = total / cnt[:, None]

    return pl.pallas_call(
        body,
        out_shape=jax.ShapeDtypeStruct((S, D), jnp.float32),
    )(partial_sums, partial_counts)


def kernel(x, batch):
    batch = batch.astype(jnp.int32)
    partial_sums, partial_counts = _sc_partials(x, batch)
    return _combine(partial_sums, partial_counts)
